# baseline (speedup 1.0000x reference)
import functools

import jax
import jax.numpy as jnp
from jax import lax
from jax.experimental import pallas as pl
from jax.experimental.pallas import tpu as pltpu
from jax.experimental.pallas import tpu_sc as plsc

N = 10000
E = 160000
HID = 512
EPS = 1e-5

NB_BKT = 64
BUCKET = 160
NPAD = NB_BKT * BUCKET
NT = 32
CHUNK = E // NT
CAP = 5120
KB = 32
ZBR = 8
REGION = 168
IDXCH = 512

@functools.cache
def _mesh():
    return plsc.VectorSubcoreMesh(
        core_axis_name="c", subcore_axis_name="s", num_cores=2, num_subcores=16
    )


def _tile_id():
    return lax.axis_index("s") * 2 + lax.axis_index("c")


def _iota16():
    return lax.broadcasted_iota(jnp.int32, (16,), 0)


@functools.cache
def _partition_kernel():
  return functools.partial(
    pl.kernel,
    out_type=(
        jax.ShapeDtypeStruct((NB_BKT, NT, CAP), jnp.int32),
        jax.ShapeDtypeStruct((NB_BKT, NT, CAP), jnp.int32),
        jax.ShapeDtypeStruct((NT * NB_BKT,), jnp.int32),
    ),
    mesh=_mesh(),
    scratch_types=[
        pltpu.VMEM((CHUNK + 16,), jnp.int32),
        pltpu.VMEM((CHUNK + 16,), jnp.int32),
        pltpu.VMEM((CHUNK + 16,), jnp.int32),
        pltpu.VMEM((CHUNK + 16,), jnp.int32),
        pltpu.VMEM((CAP + 16,), jnp.int32),
        pltpu.VMEM((CAP + 16,), jnp.int32),
        pltpu.VMEM((NB_BKT,), jnp.int32),
    ],
    compiler_params=pltpu.CompilerParams(needs_layout_passes=False),
  )(_partition_body)


def _partition_body(edge_ref, src_part, ldst_part, cnt, srcv, dstv, bktv,
                    ldstv, osrc, oldst, cntv):
    t = _tile_id()
    iota = _iota16()
    pltpu.sync_copy(edge_ref.at[pl.ds(t * CHUNK, CHUNK)],
                    srcv.at[pl.ds(0, CHUNK)])
    pltpu.sync_copy(edge_ref.at[pl.ds(E + t * CHUNK, CHUNK)],
                    dstv.at[pl.ds(0, CHUNK)])
    nv = (CHUNK + 15) // 16

    def pre(i, carry):
        d16 = dstv[pl.ds(i * 16, 16)]
        b16 = lax.div(d16, jnp.int32(BUCKET))
        bktv[pl.ds(i * 16, 16)] = b16
        ldstv[pl.ds(i * 16, 16)] = d16 - b16 * BUCKET
        return carry

    lax.fori_loop(0, nv, pre, 0)

    z16 = jnp.zeros((16,), jnp.int32)
    j16 = jnp.full((16,), BUCKET, jnp.int32)

    def per_bucket(b, carry):
        def scan(i, off):
            m = jnp.logical_and(bktv[pl.ds(i * 16, 16)] == b,
                                iota < (CHUNK - i * 16))
            mi = m.astype(jnp.int32)
            incl = plsc.cumsum(mi)
            pos = jnp.where(m, off + incl - mi, CAP + iota)
            plsc.store_scatter(osrc, [pos], srcv[pl.ds(i * 16, 16)])
            plsc.store_scatter(oldst, [pos], ldstv[pl.ds(i * 16, 16)])
            return off + incl[15]

        off = lax.fori_loop(0, nv, scan, 0)
        for p in range(KB // 16):
            osrc[pl.ds(off + p * 16, 16)] = z16
            oldst[pl.ds(off + p * 16, 16)] = j16
        pltpu.sync_copy(osrc.at[pl.ds(0, CAP)], src_part.at[b, t])
        pltpu.sync_copy(oldst.at[pl.ds(0, CAP)], ldst_part.at[b, t])
        plsc.store_scatter(cntv, [jnp.full((16,), b, jnp.int32)],
                           jnp.full((16,), off, jnp.int32))
        return carry

    lax.fori_loop(0, NB_BKT, per_bucket, 0)
    pltpu.sync_copy(cntv, cnt.at[pl.ds(t * NB_BKT, NB_BKT)])


@functools.cache
def _deg_kernel():
  return functools.partial(
    pl.kernel,
    out_type=jax.ShapeDtypeStruct((NPAD,), jnp.float32),
    mesh=_mesh(),
    scratch_types=[
        pltpu.VMEM(((BUCKET + 1) * 16,), jnp.float32),
        pltpu.VMEM((IDXCH,), jnp.int32),
        pltpu.VMEM((NT * NB_BKT + 16,), jnp.int32),
        pltpu.VMEM((BUCKET,), jnp.float32),
    ],
    compiler_params=pltpu.CompilerParams(needs_layout_passes=False),
  )(_deg_body)


def _deg_body(ldst_part, cnt, deg, hist, lbufv, cntv, degbuf):
    t = _tile_id()
    iota = _iota16()
    one0 = (iota < 1).astype(jnp.float32)
    z16 = jnp.zeros((16,), jnp.float32)
    pltpu.sync_copy(cnt, cntv.at[pl.ds(0, NT * NB_BKT)])

    for half in range(2):
        b = t + NT * half

        def zh(i, carry):
            hist[pl.ds(i * 16, 16)] = z16
            return carry

        lax.fori_loop(0, BUCKET + 1, zh, 0)

        def per_slot(tp, carry):
            c = cntv[pl.ds(tp * NB_BKT + b, 16)][0]
            nch = (c + IDXCH - 1) >> 10

            def per_chunk(k, carry2):
                pltpu.sync_copy(ldst_part.at[b, tp, pl.ds(k * IDXCH, IDXCH)],
                                lbufv)
                nv = jnp.minimum((c - k * IDXCH + 15) >> 4, IDXCH // 16)

                def per_v(i, carry3):
                    l16 = lbufv[pl.ds(i * 16, 16)]
                    for j in range(16):
                        ld = l16[j]
                        hist[pl.ds(ld * 16, 16)] = (
                            hist[pl.ds(ld * 16, 16)] + one0)
                    return carry3

                lax.fori_loop(0, nv, per_v, 0)
                return carry2

            lax.fori_loop(0, nch, per_chunk, 0)
            return carry

        lax.fori_loop(0, NT, per_slot, 0)

        def extract(r0, carry):
            idx = (iota + r0 * 16) * 16
            degbuf[pl.ds(r0 * 16, 16)] = plsc.load_gather(hist, [idx]) + 1.0
            return carry

        lax.fori_loop(0, BUCKET // 16, extract, 0)
        pltpu.sync_copy(degbuf, deg.at[pl.ds(b * BUCKET, BUCKET)])


@functools.cache
def _aggregate_kernel():
  return functools.partial(
    pl.kernel,
    out_type=jax.ShapeDtypeStruct((NPAD, HID), jnp.float32),
    mesh=_mesh(),
    scratch_types=[
        pltpu.VMEM((BUCKET + 1, HID), jnp.float32),
        pltpu.VMEM((2, KB, HID), jnp.float32),
        pltpu.VMEM((IDXCH,), jnp.int32),
        pltpu.VMEM((IDXCH // KB, KB), jnp.int32),
        pltpu.VMEM((NT * NB_BKT + 16,), jnp.int32),
        pltpu.SemaphoreType.DMA,
        pltpu.SemaphoreType.DMA,
    ],
    compiler_params=pltpu.CompilerParams(needs_layout_passes=False),
  )(_aggregate_body)


def _aggregate_body(hs, src_part, ldst_part4, cnt, agg, accum, rowbuf, idxv,
                    ldstv2, cntv, gsem0, gsem1):
    t = _tile_id()
    z16 = jnp.zeros((16,), jnp.float32)
    pltpu.sync_copy(cnt, cntv.at[pl.ds(0, NT * NB_BKT)])
    nrow_ch = IDXCH // KB

    for half in range(2):
        b = t + NT * half

        def za(r, carry):
            for cg in range(HID // 16):
                accum[r, pl.ds(cg * 16, 16)] = z16
            return carry

        lax.fori_loop(0, BUCKET + 1, za, 0)

        def per_slot(tp, carry):
            c = cntv[pl.ds(tp * NB_BKT + b, 16)][0]
            nch = (c + IDXCH - 1) >> 9

            def per_chunk(k, carry2):
                pltpu.sync_copy(src_part.at[b, tp, pl.ds(k * IDXCH, IDXCH)],
                                idxv)
                pltpu.sync_copy(
                    ldst_part4.at[b, tp, pl.ds(k * nrow_ch, nrow_ch)], ldstv2)
                nbatch = jnp.minimum((c - k * IDXCH + KB - 1) >> 5, nrow_ch)

                def gath(i, slot):
                    return pltpu.make_async_copy(
                        hs.at[idxv.at[pl.ds(i * KB, KB)]],
                        rowbuf.at[slot], gsem0 if slot == 0 else gsem1)

                @pl.when(nbatch > 0)
                def _():
                    gath(0, 0).start()

                def body(i, carry3):
                    slot = i & 1

                    @pl.when(slot == 0)
                    def _():
                        gath(i, 0).wait()

                    @pl.when(slot == 1)
                    def _():
                        gath(i, 1).wait()

                    @pl.when(i + 1 < nbatch)
                    def _():
                        @pl.when(slot == 0)
                        def _():
                            gath(i + 1, 1).start()

                        @pl.when(slot == 1)
                        def _():
                            gath(i + 1, 0).start()

                    for g in range(KB // 16):
                        l16 = ldstv2[i, pl.ds(g * 16, 16)]
                        for j in range(16):
                            ld = l16[j]
                            e = g * 16 + j

                            def cgrp(cg, carry4):
                                s = cg * 128
                                for u in range(8):
                                    accum[ld, pl.ds(s + u * 16, 16)] = (
                                        accum[ld, pl.ds(s + u * 16, 16)]
                                        + rowbuf[slot, e, pl.ds(s + u * 16, 16)])
                                return carry4

                            lax.fori_loop(0, HID // 128, cgrp, 0)
                    return carry3

                lax.fori_loop(0, nbatch, body, 0)
                return carry2

            lax.fori_loop(0, nch, per_chunk, 0)
            return carry

        lax.fori_loop(0, NT, per_slot, 0)
        pltpu.sync_copy(accum.at[pl.ds(0, BUCKET)],
                        agg.at[pl.ds(b * BUCKET, BUCKET)])


_BM = 1024
_GRID = (NPAD // _BM,)


def _mm_bias_body(x_ref, w_ref, b_ref, o_ref):
    o_ref[...] = jnp.dot(x_ref[...], w_ref[...],
                         preferred_element_type=jnp.float32) + b_ref[...]


def _tc_in_proj(x, W, bvec):
    kd, f = W.shape
    return pl.pallas_call(
        _mm_bias_body,
        grid=_GRID,
        in_specs=[
            pl.BlockSpec((_BM, kd), lambda i: (i, 0)),
            pl.BlockSpec((kd, f), lambda i: (0, 0)),
            pl.BlockSpec((1, f), lambda i: (0, 0)),
        ],
        out_specs=pl.BlockSpec((_BM, f), lambda i: (i, 0)),
        out_shape=jax.ShapeDtypeStruct((N, f), jnp.float32),
    )(x, W, bvec.reshape(1, f))


def _mm_scale_body(h_ref, w_ref, d_ref, o_ref):
    dinv = lax.rsqrt(d_ref[...])
    o_ref[...] = jnp.dot(h_ref[...], w_ref[...],
                         preferred_element_type=jnp.float32) * dinv


def _tc_mm_scale(h, W, deg2d):
    kd, f = W.shape
    return pl.pallas_call(
        _mm_scale_body,
        grid=_GRID,
        in_specs=[
            pl.BlockSpec((_BM, kd), lambda i: (i, 0)),
            pl.BlockSpec((kd, f), lambda i: (0, 0)),
            pl.BlockSpec((_BM, 1), lambda i: (i, 0)),
        ],
        out_specs=pl.BlockSpec((_BM, f), lambda i: (i, 0)),
        out_shape=jax.ShapeDtypeStruct((NPAD, f), jnp.float32),
    )(h, W, deg2d)


def _ln_res_body(h_ref, a_ref, s_ref, d_ref, b_ref, g_ref, lb_ref, o_ref):
    dinv = lax.rsqrt(d_ref[...])
    y = (a_ref[...] + s_ref[...]) * dinv + b_ref[...]
    mu = jnp.mean(y, axis=-1, keepdims=True)
    yc = y - mu
    var = jnp.mean(yc * yc, axis=-1, keepdims=True)
    y = yc * lax.rsqrt(var + EPS) * g_ref[...] + lb_ref[...]
    o_ref[...] = h_ref[...] + jnp.maximum(y, 0.0)


def _tc_ln_res(h, agg, hs, deg2d, bvec, gvec, lbvec):
    f = h.shape[1]
    vec = pl.BlockSpec((1, f), lambda i: (0, 0))
    blk = pl.BlockSpec((_BM, f), lambda i: (i, 0))
    return pl.pallas_call(
        _ln_res_body,
        grid=_GRID,
        in_specs=[blk, blk, blk, pl.BlockSpec((_BM, 1), lambda i: (i, 0)),
                  vec, vec, vec],
        out_specs=blk,
        out_shape=jax.ShapeDtypeStruct((N, f), jnp.float32),
    )(h, agg, hs, deg2d, bvec.reshape(1, f), gvec.reshape(1, f),
      lbvec.reshape(1, f))


def _mm_body(x_ref, w_ref, o_ref):
    o_ref[...] = jnp.dot(x_ref[...], w_ref[...],
                         preferred_element_type=jnp.float32)


def _tc_out_proj(h, W):
    kd, f = W.shape
    return pl.pallas_call(
        _mm_body,
        grid=_GRID,
        in_specs=[
            pl.BlockSpec((_BM, kd), lambda i: (i, 0)),
            pl.BlockSpec((kd, f), lambda i: (0, 0)),
        ],
        out_specs=pl.BlockSpec((_BM, f), lambda i: (i, 0)),
        out_shape=jax.ShapeDtypeStruct((N, f), jnp.float32),
    )(h, W)


def kernel(x, edge_index, W_in, b_in, gcn_W, gcn_b, ln_g, ln_b, W_out):
    edge_flat = edge_index.astype(jnp.int32).reshape(2 * E)
    src_part, ldst_part, cnt = _partition_kernel()(edge_flat)
    deg = _deg_kernel()(ldst_part, cnt)
    deg2d = deg.reshape(NPAD, 1)
    h = _tc_in_proj(x, W_in, b_in)
    ldst_part4 = ldst_part.reshape(NB_BKT, NT, CAP // KB, KB)
    for i in range(3):
        hs = _tc_mm_scale(h, gcn_W[i], deg2d)
        agg = _aggregate_kernel()(hs, src_part, ldst_part4, cnt)
        h = _tc_ln_res(h, agg, hs, deg2d, gcn_b[i], ln_g[i], ln_b[i])
    return _tc_out_proj(h, W_out)


# speedup vs baseline: 1.0555x; 1.0555x over previous
import functools

import jax
import jax.numpy as jnp
from jax import lax
from jax.experimental import pallas as pl
from jax.experimental.pallas import tpu as pltpu
from jax.experimental.pallas import tpu_sc as plsc

N = 10000
E = 160000
HID = 512
EPS = 1e-5

NB_BKT = 64
BUCKET = 160
NPAD = NB_BKT * BUCKET
NT = 32
CHUNK = E // NT
CAP = 5120
KB = 32
ZBR = 8
REGION = 168
IDXCH = 512
IDXSH = 9

@functools.cache
def _mesh():
    return plsc.VectorSubcoreMesh(
        core_axis_name="c", subcore_axis_name="s", num_cores=2, num_subcores=16
    )


def _tile_id():
    return lax.axis_index("s") * 2 + lax.axis_index("c")


def _iota16():
    return lax.broadcasted_iota(jnp.int32, (16,), 0)


@functools.cache
def _partition_kernel():
  return functools.partial(
    pl.kernel,
    out_type=(
        jax.ShapeDtypeStruct((NB_BKT * NT * CAP,), jnp.int32),
        jax.ShapeDtypeStruct((NB_BKT * NT * CAP,), jnp.int32),
        jax.ShapeDtypeStruct((NT * NB_BKT,), jnp.int32),
    ),
    mesh=_mesh(),
    scratch_types=[
        pltpu.VMEM((CHUNK + 16,), jnp.int32),
        pltpu.VMEM((CHUNK + 16,), jnp.int32),
        pltpu.VMEM((CHUNK + 16,), jnp.int32),
        pltpu.VMEM((CHUNK + 16,), jnp.int32),
        pltpu.VMEM((CAP + 16,), jnp.int32),
        pltpu.VMEM((CAP + 16,), jnp.int32),
        pltpu.VMEM((NB_BKT,), jnp.int32),
    ],
    compiler_params=pltpu.CompilerParams(needs_layout_passes=False),
  )(_partition_body)


def _partition_body(edge_ref, src_part, ldst_part, cnt, srcv, dstv, bktv,
                    ldstv, osrc, oldst, cntv):
    t = _tile_id()
    iota = _iota16()
    pltpu.sync_copy(edge_ref.at[pl.ds(t * CHUNK, CHUNK)],
                    srcv.at[pl.ds(0, CHUNK)])
    pltpu.sync_copy(edge_ref.at[pl.ds(E + t * CHUNK, CHUNK)],
                    dstv.at[pl.ds(0, CHUNK)])
    nv = (CHUNK + 15) // 16

    def pre(i, carry):
        d16 = dstv[pl.ds(i * 16, 16)]
        b16 = lax.div(d16, jnp.int32(BUCKET))
        bktv[pl.ds(i * 16, 16)] = b16
        ldstv[pl.ds(i * 16, 16)] = d16 - b16 * BUCKET
        return carry

    lax.fori_loop(0, nv, pre, 0)

    z16 = jnp.zeros((16,), jnp.int32)
    j16 = jnp.full((16,), BUCKET, jnp.int32)

    def per_bucket(b, carry):
        def scan(i, off):
            m = jnp.logical_and(bktv[pl.ds(i * 16, 16)] == b,
                                iota < (CHUNK - i * 16))
            mi = m.astype(jnp.int32)
            incl = plsc.cumsum(mi)
            pos = jnp.where(m, off + incl - mi, CAP + iota)
            plsc.store_scatter(osrc, [pos], srcv[pl.ds(i * 16, 16)])
            plsc.store_scatter(oldst, [pos], ldstv[pl.ds(i * 16, 16)])
            return off + incl[15]

        off = lax.fori_loop(0, nv, scan, 0)
        for p in range(KB // 16):
            osrc[pl.ds(off + p * 16, 16)] = z16
            oldst[pl.ds(off + p * 16, 16)] = j16
        slot0 = (b * NT + t) * CAP
        pltpu.sync_copy(osrc.at[pl.ds(0, CAP)], src_part.at[pl.ds(slot0, CAP)])
        pltpu.sync_copy(oldst.at[pl.ds(0, CAP)],
                        ldst_part.at[pl.ds(slot0, CAP)])
        plsc.store_scatter(cntv, [jnp.full((16,), b, jnp.int32)],
                           jnp.full((16,), off, jnp.int32))
        return carry

    lax.fori_loop(0, NB_BKT, per_bucket, 0)
    pltpu.sync_copy(cntv, cnt.at[pl.ds(t * NB_BKT, NB_BKT)])


@functools.cache
def _deg_kernel():
  return functools.partial(
    pl.kernel,
    out_type=jax.ShapeDtypeStruct((NPAD,), jnp.float32),
    mesh=_mesh(),
    scratch_types=[
        pltpu.VMEM(((BUCKET + 1) * 16,), jnp.float32),
        pltpu.VMEM((IDXCH,), jnp.int32),
        pltpu.VMEM((NT * NB_BKT + 16,), jnp.int32),
        pltpu.VMEM((BUCKET,), jnp.float32),
    ],
    compiler_params=pltpu.CompilerParams(needs_layout_passes=False),
  )(_deg_body)


def _deg_body(ldst_part, cnt, deg, hist, lbufv, cntv, degbuf):
    t = _tile_id()
    iota = _iota16()
    one0 = (iota < 1).astype(jnp.float32)
    z16 = jnp.zeros((16,), jnp.float32)
    pltpu.sync_copy(cnt, cntv.at[pl.ds(0, NT * NB_BKT)])

    for half in range(2):
        b = t + NT * half

        def zh(i, carry):
            hist[pl.ds(i * 16, 16)] = z16
            return carry

        lax.fori_loop(0, BUCKET + 1, zh, 0)

        def per_slot(tp, carry):
            c = cntv[pl.ds(tp * NB_BKT + b, 16)][0]
            nch = (c + IDXCH - 1) >> IDXSH

            def per_chunk(k, carry2):
                pltpu.sync_copy(
                    ldst_part.at[pl.ds((b * NT + tp) * CAP + k * IDXCH,
                                       IDXCH)], lbufv)
                nv = jnp.minimum((c - k * IDXCH + 15) >> 4, IDXCH // 16)

                def per_v(i, carry3):
                    l16 = lbufv[pl.ds(i * 16, 16)]
                    for j in range(16):
                        ld = l16[j]
                        hist[pl.ds(ld * 16, 16)] = (
                            hist[pl.ds(ld * 16, 16)] + one0)
                    return carry3

                lax.fori_loop(0, nv, per_v, 0)
                return carry2

            lax.fori_loop(0, nch, per_chunk, 0)
            return carry

        lax.fori_loop(0, NT, per_slot, 0)

        def extract(r0, carry):
            idx = (iota + r0 * 16) * 16
            degbuf[pl.ds(r0 * 16, 16)] = plsc.load_gather(hist, [idx]) + 1.0
            return carry

        lax.fori_loop(0, BUCKET // 16, extract, 0)
        pltpu.sync_copy(degbuf, deg.at[pl.ds(b * BUCKET, BUCKET)])


@functools.cache
def _aggregate_kernel():
  return functools.partial(
    pl.kernel,
    out_type=jax.ShapeDtypeStruct((NPAD, HID), jnp.float32),
    mesh=_mesh(),
    scratch_types=[
        pltpu.VMEM((BUCKET + 1, HID), jnp.float32),
        pltpu.VMEM((2, KB, HID), jnp.float32),
        pltpu.VMEM((2, IDXCH), jnp.int32),
        pltpu.VMEM((2, IDXCH), jnp.int32),
        pltpu.VMEM((NT * NB_BKT + 16,), jnp.int32),
        pltpu.SemaphoreType.DMA,
        pltpu.SemaphoreType.DMA,
        pltpu.SemaphoreType.DMA,
    ],
    compiler_params=pltpu.CompilerParams(needs_layout_passes=False),
  )(_aggregate_body)


def _aggregate_body(hs, src_part, ldst_part, cnt, agg, accum, rowbuf, idxv,
                    ldstv, cntv, gsem0, gsem1, stsem):
    t = _tile_id()
    z16 = jnp.zeros((16,), jnp.float32)
    pltpu.sync_copy(cnt, cntv.at[pl.ds(0, NT * NB_BKT)])

    for half in range(2):
        b = t + NT * half

        def slot_base(tp):
            return (b * NT + tp) * CAP

        def stage(tp, sbuf):
            pltpu.make_async_copy(
                src_part.at[pl.ds(slot_base(tp), IDXCH)],
                idxv.at[sbuf], stsem).start()
            pltpu.make_async_copy(
                ldst_part.at[pl.ds(slot_base(tp), IDXCH)],
                ldstv.at[sbuf], stsem).start()

        def stage_wait(sbuf):
            pltpu.make_async_copy(
                src_part.at[pl.ds(0, IDXCH)], idxv.at[sbuf], stsem).wait()
            pltpu.make_async_copy(
                ldst_part.at[pl.ds(0, IDXCH)], ldstv.at[sbuf], stsem).wait()

        def za(r, carry):
            for cg in range(HID // 16):
                accum[r, pl.ds(cg * 16, 16)] = z16
            return carry

        lax.fori_loop(0, BUCKET + 1, za, 0)
        stage(0, 0)

        def per_slot(tp, carry):
            sbuf = tp & 1
            c = cntv[pl.ds(tp * NB_BKT + b, 16)][0]

            @pl.when(sbuf == 0)
            def _():
                stage_wait(0)

                @pl.when(tp + 1 < NT)
                def _():
                    stage(tp + 1, 1)

            @pl.when(sbuf == 1)
            def _():
                stage_wait(1)

                @pl.when(tp + 1 < NT)
                def _():
                    stage(tp + 1, 0)

            nch = (c + IDXCH - 1) >> IDXSH

            def per_chunk(k, carry2):
                @pl.when(k > 0)
                def _():
                    pltpu.sync_copy(
                        src_part.at[pl.ds(slot_base(tp) + k * IDXCH, IDXCH)],
                        idxv.at[sbuf])
                    pltpu.sync_copy(
                        ldst_part.at[pl.ds(slot_base(tp) + k * IDXCH, IDXCH)],
                        ldstv.at[sbuf])

                nbatch = jnp.minimum((c - k * IDXCH + KB - 1) >> 5,
                                     IDXCH // KB)

                def gath(i, slot):
                    return pltpu.make_async_copy(
                        hs.at[idxv.at[sbuf, pl.ds(i * KB, KB)]],
                        rowbuf.at[slot], gsem0 if slot == 0 else gsem1)

                @pl.when(nbatch > 0)
                def _():
                    gath(0, 0).start()

                def body(i, carry3):
                    slot = i & 1

                    @pl.when(slot == 0)
                    def _():
                        gath(i, 0).wait()

                    @pl.when(slot == 1)
                    def _():
                        gath(i, 1).wait()

                    @pl.when(i + 1 < nbatch)
                    def _():
                        @pl.when(slot == 0)
                        def _():
                            gath(i + 1, 1).start()

                        @pl.when(slot == 1)
                        def _():
                            gath(i + 1, 0).start()

                    for g in range(KB // 16):
                        l16 = ldstv[sbuf, pl.ds(i * KB + g * 16, 16)]
                        for j in range(16):
                            ld = l16[j]
                            e = g * 16 + j

                            def cgrp(cg, carry4):
                                cs = cg * 128
                                for u in range(8):
                                    accum[ld, pl.ds(cs + u * 16, 16)] = (
                                        accum[ld, pl.ds(cs + u * 16, 16)]
                                        + rowbuf[slot, e, pl.ds(cs + u * 16, 16)])
                                return carry4

                            lax.fori_loop(0, HID // 128, cgrp, 0)
                    return carry3

                lax.fori_loop(0, nbatch, body, 0)
                return carry2

            lax.fori_loop(0, nch, per_chunk, 0)
            return carry

        lax.fori_loop(0, NT, per_slot, 0)
        pltpu.sync_copy(accum.at[pl.ds(0, BUCKET)],
                        agg.at[pl.ds(b * BUCKET, BUCKET)])


_BM = 1024
_GRID = (NPAD // _BM,)


def _mm_bias_body(x_ref, w_ref, b_ref, o_ref):
    o_ref[...] = jnp.dot(x_ref[...], w_ref[...],
                         preferred_element_type=jnp.float32) + b_ref[...]


def _tc_in_proj(x, W, bvec):
    kd, f = W.shape
    return pl.pallas_call(
        _mm_bias_body,
        grid=_GRID,
        in_specs=[
            pl.BlockSpec((_BM, kd), lambda i: (i, 0)),
            pl.BlockSpec((kd, f), lambda i: (0, 0)),
            pl.BlockSpec((1, f), lambda i: (0, 0)),
        ],
        out_specs=pl.BlockSpec((_BM, f), lambda i: (i, 0)),
        out_shape=jax.ShapeDtypeStruct((N, f), jnp.float32),
    )(x, W, bvec.reshape(1, f))


def _mm_scale_body(h_ref, w_ref, d_ref, o_ref):
    dinv = lax.rsqrt(d_ref[...])
    o_ref[...] = jnp.dot(h_ref[...], w_ref[...],
                         preferred_element_type=jnp.float32) * dinv


def _tc_mm_scale(h, W, deg2d):
    kd, f = W.shape
    return pl.pallas_call(
        _mm_scale_body,
        grid=_GRID,
        in_specs=[
            pl.BlockSpec((_BM, kd), lambda i: (i, 0)),
            pl.BlockSpec((kd, f), lambda i: (0, 0)),
            pl.BlockSpec((_BM, 1), lambda i: (i, 0)),
        ],
        out_specs=pl.BlockSpec((_BM, f), lambda i: (i, 0)),
        out_shape=jax.ShapeDtypeStruct((NPAD, f), jnp.float32),
    )(h, W, deg2d)


def _ln_res_body(h_ref, a_ref, s_ref, d_ref, b_ref, g_ref, lb_ref, o_ref):
    dinv = lax.rsqrt(d_ref[...])
    y = (a_ref[...] + s_ref[...]) * dinv + b_ref[...]
    mu = jnp.mean(y, axis=-1, keepdims=True)
    yc = y - mu
    var = jnp.mean(yc * yc, axis=-1, keepdims=True)
    y = yc * lax.rsqrt(var + EPS) * g_ref[...] + lb_ref[...]
    o_ref[...] = h_ref[...] + jnp.maximum(y, 0.0)


def _tc_ln_res(h, agg, hs, deg2d, bvec, gvec, lbvec):
    f = h.shape[1]
    vec = pl.BlockSpec((1, f), lambda i: (0, 0))
    blk = pl.BlockSpec((_BM, f), lambda i: (i, 0))
    return pl.pallas_call(
        _ln_res_body,
        grid=_GRID,
        in_specs=[blk, blk, blk, pl.BlockSpec((_BM, 1), lambda i: (i, 0)),
                  vec, vec, vec],
        out_specs=blk,
        out_shape=jax.ShapeDtypeStruct((N, f), jnp.float32),
    )(h, agg, hs, deg2d, bvec.reshape(1, f), gvec.reshape(1, f),
      lbvec.reshape(1, f))


def _mm_body(x_ref, w_ref, o_ref):
    o_ref[...] = jnp.dot(x_ref[...], w_ref[...],
                         preferred_element_type=jnp.float32)


def _tc_out_proj(h, W):
    kd, f = W.shape
    return pl.pallas_call(
        _mm_body,
        grid=_GRID,
        in_specs=[
            pl.BlockSpec((_BM, kd), lambda i: (i, 0)),
            pl.BlockSpec((kd, f), lambda i: (0, 0)),
        ],
        out_specs=pl.BlockSpec((_BM, f), lambda i: (i, 0)),
        out_shape=jax.ShapeDtypeStruct((N, f), jnp.float32),
    )(h, W)


def kernel(x, edge_index, W_in, b_in, gcn_W, gcn_b, ln_g, ln_b, W_out):
    edge_flat = edge_index.astype(jnp.int32).reshape(2 * E)
    src_part, ldst_part, cnt = _partition_kernel()(edge_flat)
    deg = _deg_kernel()(ldst_part, cnt)
    deg2d = deg.reshape(NPAD, 1)
    h = _tc_in_proj(x, W_in, b_in)
    for i in range(3):
        hs = _tc_mm_scale(h, gcn_W[i], deg2d)
        agg = _aggregate_kernel()(hs, src_part, ldst_part, cnt)
        h = _tc_ln_res(h, agg, hs, deg2d, gcn_b[i], ln_g[i], ln_b[i])
    return _tc_out_proj(h, W_out)
